# baseline (speedup 1.0000x reference)
import functools

import jax
import jax.numpy as jnp
from jax import lax
from jax.experimental import pallas as pl
from jax.experimental.pallas import tpu as pltpu
from jax.experimental.pallas import tpu_sc as plsc

NC = 2
NS = 16
NW = NC * NS
DEPTH = 4


def _mesh():
    return plsc.VectorSubcoreMesh(
        core_axis_name="c", subcore_axis_name="s", num_cores=NC, num_subcores=NS
    )


def _make_sc_count(E, n_pad, slice_len, chunk):
    epw = E // NW
    nchunks = epw // chunk

    @functools.partial(
        pl.kernel,
        out_type=jax.ShapeDtypeStruct((NC * n_pad,), jnp.float32),
        mesh=_mesh(),
        scratch_types=(
            [pltpu.VMEM((chunk,), jnp.int32) for _ in range(DEPTH)]
            + [
                pltpu.VMEM((chunk,), jnp.float32),
                pltpu.VMEM((slice_len,), jnp.float32),
                pltpu.VMEM_SHARED((n_pad,), jnp.float32),
            ]
            + [pltpu.SemaphoreType.DMA for _ in range(DEPTH)]
        ),
    )
    def count_kernel(edge_hbm, zeros_hbm, ones_hbm, out_hbm, *scratch):
        didx = scratch[:DEPTH]
        onesv, stagev, accsh = scratch[DEPTH:DEPTH + 3]
        sems = scratch[DEPTH + 3:]
        cid = lax.axis_index("c")
        sid = lax.axis_index("s")
        wid = sid * NC + cid
        pltpu.sync_copy(zeros_hbm, stagev)
        pltpu.sync_copy(stagev, accsh.at[pl.ds(sid * slice_len, slice_len)])
        pltpu.sync_copy(ones_hbm, onesv)
        plsc.subcore_barrier()
        base = E + wid * epw
        pending = [None] * DEPTH
        for i in range(nchunks):
            r = i % DEPTH
            if pending[r] is not None:
                pending[r].wait()
            pltpu.sync_copy(edge_hbm.at[pl.ds(base + i * chunk, chunk)], didx[r])
            pending[r] = pltpu.async_copy(onesv, accsh.at[didx[r]], sems[r], add=True)
        for p in pending:
            if p is not None:
                p.wait()
        plsc.subcore_barrier()
        pltpu.sync_copy(accsh.at[pl.ds(sid * slice_len, slice_len)], stagev)
        pltpu.sync_copy(stagev, out_hbm.at[pl.ds(cid * n_pad + sid * slice_len, slice_len)])

    return count_kernel


def _make_sc_gather_scatter(E, n_pad, slice_len, chunk):
    epw = E // NW
    nchunks = epw // chunk

    @functools.partial(
        pl.kernel,
        out_type=jax.ShapeDtypeStruct((NC * n_pad,), jnp.float32),
        mesh=_mesh(),
        scratch_types=(
            [pltpu.VMEM((chunk,), jnp.int32) for _ in range(2 * DEPTH)]
            + [pltpu.VMEM((chunk,), jnp.float32) for _ in range(DEPTH)]
            + [
                pltpu.VMEM((slice_len,), jnp.float32),
                pltpu.VMEM_SHARED((n_pad,), jnp.float32),
            ]
            + [pltpu.SemaphoreType.DMA for _ in range(2 * DEPTH)]
        ),
    )
    def gs_kernel(edge_hbm, table_hbm, zeros_hbm, out_hbm, *scratch):
        sidx = scratch[:DEPTH]
        didx = scratch[DEPTH:2 * DEPTH]
        vals = scratch[2 * DEPTH:3 * DEPTH]
        stagev, accsh = scratch[3 * DEPTH:3 * DEPTH + 2]
        semg = scratch[3 * DEPTH + 2:3 * DEPTH + 2 + DEPTH]
        sems = scratch[3 * DEPTH + 2 + DEPTH:]
        cid = lax.axis_index("c")
        sid = lax.axis_index("s")
        wid = sid * NC + cid
        sl = pl.ds(sid * slice_len, slice_len)
        pltpu.sync_copy(zeros_hbm, stagev)
        pltpu.sync_copy(stagev, accsh.at[sl])
        plsc.subcore_barrier()
        base = wid * epw
        gat = [None] * DEPTH
        pending = [None] * DEPTH
        for i in range(nchunks):
            r = i % DEPTH
            if pending[r] is not None:
                pending[r].wait()
                pending[r] = None
            off = base + i * chunk
            pltpu.sync_copy(edge_hbm.at[pl.ds(off, chunk)], sidx[r])
            pltpu.sync_copy(edge_hbm.at[pl.ds(E + off, chunk)], didx[r])
            gat[r] = pltpu.async_copy(table_hbm.at[sidx[r]], vals[r], semg[r])
            if i >= 1:
                rp = (i - 1) % DEPTH
                gat[rp].wait()
                gat[rp] = None
                pending[rp] = pltpu.async_copy(
                    vals[rp], accsh.at[didx[rp]], sems[rp], add=True)
        rl = (nchunks - 1) % DEPTH
        gat[rl].wait()
        pending[rl] = pltpu.async_copy(vals[rl], accsh.at[didx[rl]], sems[rl], add=True)
        for p in pending:
            if p is not None:
                p.wait()
        plsc.subcore_barrier()
        pltpu.sync_copy(accsh.at[sl], stagev)
        pltpu.sync_copy(stagev, out_hbm.at[pl.ds(cid * n_pad + sid * slice_len, slice_len)])

    return gs_kernel


def _tc1_body(cnt_ref, xp_ref, dinv_ref, u1_ref):
    deg = cnt_ref[0] + cnt_ref[1] + 1.0
    dinv = lax.rsqrt(deg)
    dinv_ref[...] = dinv
    u1_ref[...] = dinv * xp_ref[...]


def _tc2_body(t1_ref, dinv_ref, xp_ref, w1_ref, b1_ref, w2_ref, y_ref, u2_ref):
    dinv = dinv_ref[...]
    s = dinv * (t1_ref[0] + t1_ref[1]) + dinv * dinv * xp_ref[...]
    y = jnp.zeros_like(s)
    for j in range(16):
        y = y + jnp.maximum(s * w1_ref[0, j] + b1_ref[0, j], 0.0) * w2_ref[0, j]
    y_ref[...] = y
    u2_ref[...] = dinv * y


def _tc3_body(t2_ref, dinv_ref, y_ref, b2_ref, out_ref):
    dinv = dinv_ref[...]
    out_ref[...] = dinv * (t2_ref[0] + t2_ref[1]) + dinv * dinv * y_ref[...] + b2_ref[0, 0]


def kernel(x, edge_index, W1, b1, W2, b2):
    N = x.shape[0]
    E = edge_index.shape[1]
    assert E % NW == 0
    epw = E // NW
    chunk_cnt = 10000
    chunk_gs = 5000
    assert epw % chunk_cnt == 0 and epw % chunk_gs == 0
    slice_len = (-(-N // NS) + 7) // 8 * 8
    n_pad = slice_len * NS
    rows = n_pad // 128

    sc_count = _make_sc_count(E, n_pad, slice_len, chunk_cnt)
    sc_gs = _make_sc_gather_scatter(E, n_pad, slice_len, chunk_gs)

    zeros_s = jnp.zeros((slice_len,), jnp.float32)
    ones_c = jnp.ones((chunk_cnt,), jnp.float32)
    xp = jnp.pad(x[:, 0], (0, n_pad - N)).reshape(rows, 128)

    ei_flat = edge_index.reshape(2 * E)
    cnt = sc_count(ei_flat, zeros_s, ones_c)

    dinv, u1 = pl.pallas_call(
        _tc1_body,
        out_shape=[
            jax.ShapeDtypeStruct((rows, 128), jnp.float32),
            jax.ShapeDtypeStruct((rows, 128), jnp.float32),
        ],
    )(cnt.reshape(2, rows, 128), xp)

    t1 = sc_gs(ei_flat, u1.reshape(n_pad), zeros_s)

    y, u2 = pl.pallas_call(
        _tc2_body,
        out_shape=[
            jax.ShapeDtypeStruct((rows, 128), jnp.float32),
            jax.ShapeDtypeStruct((rows, 128), jnp.float32),
        ],
    )(t1.reshape(2, rows, 128), dinv, xp, W1, b1.reshape(1, 16), W2.reshape(1, 16))

    t2 = sc_gs(ei_flat, u2.reshape(n_pad), zeros_s)

    out = pl.pallas_call(
        _tc3_body,
        out_shape=jax.ShapeDtypeStruct((rows, 128), jnp.float32),
    )(t2.reshape(2, rows, 128), dinv, y, b2.reshape(1, 1))

    return out.reshape(n_pad)[:N].reshape(N, 1)


# speedup vs baseline: 1.7417x; 1.7417x over previous
import functools

import jax
import jax.numpy as jnp
from jax import lax
from jax.experimental import pallas as pl
from jax.experimental.pallas import tpu as pltpu
from jax.experimental.pallas import tpu_sc as plsc

NC = 2
NS = 16
NW = NC * NS
DEPTH = 8


def _mesh():
    return plsc.VectorSubcoreMesh(
        core_axis_name="c", subcore_axis_name="s", num_cores=NC, num_subcores=NS
    )


def _make_sc_count(E, n_pad, slice_len, chunk):
    epw = E // NW
    nchunks = epw // chunk

    @functools.partial(
        pl.kernel,
        out_type=jax.ShapeDtypeStruct((NC * n_pad,), jnp.float32),
        mesh=_mesh(),
        scratch_types=(
            [pltpu.VMEM((chunk,), jnp.int32) for _ in range(DEPTH)]
            + [
                pltpu.VMEM((chunk,), jnp.float32),
                pltpu.VMEM((slice_len,), jnp.float32),
                pltpu.VMEM_SHARED((n_pad,), jnp.float32),
            ]
            + [pltpu.SemaphoreType.DMA for _ in range(DEPTH)]
        ),
    )
    def count_kernel(edge_hbm, zeros_hbm, ones_hbm, out_hbm, *scratch):
        didx = scratch[:DEPTH]
        onesv, stagev, accsh = scratch[DEPTH:DEPTH + 3]
        sems = scratch[DEPTH + 3:]
        cid = lax.axis_index("c")
        sid = lax.axis_index("s")
        wid = sid * NC + cid
        pltpu.sync_copy(zeros_hbm, stagev)
        pltpu.sync_copy(stagev, accsh.at[pl.ds(sid * slice_len, slice_len)])
        pltpu.sync_copy(ones_hbm, onesv)
        plsc.subcore_barrier()
        base = E + wid * epw
        pending = [None] * DEPTH
        for i in range(nchunks):
            r = i % DEPTH
            if pending[r] is not None:
                pending[r].wait()
            pltpu.sync_copy(edge_hbm.at[pl.ds(base + i * chunk, chunk)], didx[r])
            pending[r] = pltpu.async_copy(onesv, accsh.at[didx[r]], sems[r], add=True)
        for p in pending:
            if p is not None:
                p.wait()
        plsc.subcore_barrier()
        pltpu.sync_copy(accsh.at[pl.ds(sid * slice_len, slice_len)], stagev)
        pltpu.sync_copy(stagev, out_hbm.at[pl.ds(cid * n_pad + sid * slice_len, slice_len)])

    return count_kernel


def _make_sc_gather_scatter(E, n_pad, slice_len, chunk):
    epw = E // NW
    nchunks = epw // chunk

    @functools.partial(
        pl.kernel,
        out_type=jax.ShapeDtypeStruct((NC * n_pad,), jnp.float32),
        mesh=_mesh(),
        scratch_types=(
            [pltpu.VMEM((chunk,), jnp.int32) for _ in range(2 * DEPTH)]
            + [pltpu.VMEM((chunk,), jnp.float32) for _ in range(DEPTH)]
            + [
                pltpu.VMEM((slice_len,), jnp.float32),
                pltpu.VMEM_SHARED((n_pad,), jnp.float32),
                pltpu.VMEM_SHARED((n_pad,), jnp.float32),
            ]
            + [pltpu.SemaphoreType.DMA for _ in range(2 * DEPTH)]
        ),
    )
    def gs_kernel(edge_hbm, table_hbm, zeros_hbm, out_hbm, *scratch):
        sidx = scratch[:DEPTH]
        didx = scratch[DEPTH:2 * DEPTH]
        vals = scratch[2 * DEPTH:3 * DEPTH]
        stagev, tabsh, accsh = scratch[3 * DEPTH:3 * DEPTH + 3]
        semg = scratch[3 * DEPTH + 3:3 * DEPTH + 3 + DEPTH]
        sems = scratch[3 * DEPTH + 3 + DEPTH:]
        cid = lax.axis_index("c")
        sid = lax.axis_index("s")
        wid = sid * NC + cid
        sl = pl.ds(sid * slice_len, slice_len)
        pltpu.sync_copy(zeros_hbm, stagev)
        pltpu.sync_copy(stagev, accsh.at[sl])
        pltpu.sync_copy(table_hbm.at[sl], stagev)
        pltpu.sync_copy(stagev, tabsh.at[sl])
        plsc.subcore_barrier()
        base = wid * epw
        gat = [None] * DEPTH
        pending = [None] * DEPTH
        for i in range(nchunks):
            r = i % DEPTH
            if pending[r] is not None:
                pending[r].wait()
                pending[r] = None
            off = base + i * chunk
            pltpu.sync_copy(edge_hbm.at[pl.ds(off, chunk)], sidx[r])
            pltpu.sync_copy(edge_hbm.at[pl.ds(E + off, chunk)], didx[r])
            gat[r] = pltpu.async_copy(tabsh.at[sidx[r]], vals[r], semg[r])
            if i >= 1:
                rp = (i - 1) % DEPTH
                gat[rp].wait()
                gat[rp] = None
                pending[rp] = pltpu.async_copy(
                    vals[rp], accsh.at[didx[rp]], sems[rp], add=True)
        rl = (nchunks - 1) % DEPTH
        gat[rl].wait()
        pending[rl] = pltpu.async_copy(vals[rl], accsh.at[didx[rl]], sems[rl], add=True)
        for p in pending:
            if p is not None:
                p.wait()
        plsc.subcore_barrier()
        pltpu.sync_copy(accsh.at[sl], stagev)
        pltpu.sync_copy(stagev, out_hbm.at[pl.ds(cid * n_pad + sid * slice_len, slice_len)])

    return gs_kernel


def _tc1_body(cnt_ref, xp_ref, dinv_ref, u1_ref):
    deg = cnt_ref[0] + cnt_ref[1] + 1.0
    dinv = lax.rsqrt(deg)
    dinv_ref[...] = dinv
    u1_ref[...] = dinv * xp_ref[...]


def _tc2_body(t1_ref, dinv_ref, xp_ref, w1_ref, b1_ref, w2_ref, y_ref, u2_ref):
    dinv = dinv_ref[...]
    s = dinv * (t1_ref[0] + t1_ref[1]) + dinv * dinv * xp_ref[...]
    y = jnp.zeros_like(s)
    for j in range(16):
        y = y + jnp.maximum(s * w1_ref[0, j] + b1_ref[0, j], 0.0) * w2_ref[0, j]
    y_ref[...] = y
    u2_ref[...] = dinv * y


def _tc3_body(t2_ref, dinv_ref, y_ref, b2_ref, out_ref):
    dinv = dinv_ref[...]
    out_ref[...] = dinv * (t2_ref[0] + t2_ref[1]) + dinv * dinv * y_ref[...] + b2_ref[0, 0]


def kernel(x, edge_index, W1, b1, W2, b2):
    N = x.shape[0]
    E = edge_index.shape[1]
    assert E % NW == 0
    epw = E // NW
    chunk_cnt = 5000
    chunk_gs = 4000
    assert epw % chunk_cnt == 0 and epw % chunk_gs == 0
    slice_len = (-(-N // NS) + 7) // 8 * 8
    n_pad = slice_len * NS
    rows = n_pad // 128

    sc_count = _make_sc_count(E, n_pad, slice_len, chunk_cnt)
    sc_gs = _make_sc_gather_scatter(E, n_pad, slice_len, chunk_gs)

    zeros_s = jnp.zeros((slice_len,), jnp.float32)
    ones_c = jnp.ones((chunk_cnt,), jnp.float32)
    xp = jnp.pad(x[:, 0], (0, n_pad - N)).reshape(rows, 128)

    ei_flat = edge_index.reshape(2 * E)
    cnt = sc_count(ei_flat, zeros_s, ones_c)

    dinv, u1 = pl.pallas_call(
        _tc1_body,
        out_shape=[
            jax.ShapeDtypeStruct((rows, 128), jnp.float32),
            jax.ShapeDtypeStruct((rows, 128), jnp.float32),
        ],
    )(cnt.reshape(2, rows, 128), xp)

    t1 = sc_gs(ei_flat, u1.reshape(n_pad), zeros_s)

    y, u2 = pl.pallas_call(
        _tc2_body,
        out_shape=[
            jax.ShapeDtypeStruct((rows, 128), jnp.float32),
            jax.ShapeDtypeStruct((rows, 128), jnp.float32),
        ],
    )(t1.reshape(2, rows, 128), dinv, xp, W1, b1.reshape(1, 16), W2.reshape(1, 16))

    t2 = sc_gs(ei_flat, u2.reshape(n_pad), zeros_s)

    out = pl.pallas_call(
        _tc3_body,
        out_shape=jax.ShapeDtypeStruct((rows, 128), jnp.float32),
    )(t2.reshape(2, rows, 128), dinv, y, b2.reshape(1, 1))

    return out.reshape(n_pad)[:N].reshape(N, 1)


# speedup vs baseline: 1.7576x; 1.0092x over previous
import functools

import jax
import jax.numpy as jnp
from jax import lax
from jax.experimental import pallas as pl
from jax.experimental.pallas import tpu as pltpu
from jax.experimental.pallas import tpu_sc as plsc

NC = 2
NS = 16
NW = NC * NS
DEPTH = 4


def _mesh():
    return plsc.VectorSubcoreMesh(
        core_axis_name="c", subcore_axis_name="s", num_cores=NC, num_subcores=NS
    )


def _make_sc_count(E, n_pad, slice_len, chunk):
    epw = E // NW
    nchunks = epw // chunk

    @functools.partial(
        pl.kernel,
        out_type=jax.ShapeDtypeStruct((NC * n_pad,), jnp.float32),
        mesh=_mesh(),
        scratch_types=(
            [pltpu.VMEM((chunk,), jnp.int32) for _ in range(DEPTH)]
            + [
                pltpu.VMEM((chunk,), jnp.float32),
                pltpu.VMEM((slice_len,), jnp.float32),
                pltpu.VMEM_SHARED((n_pad,), jnp.float32),
            ]
            + [pltpu.SemaphoreType.DMA for _ in range(DEPTH)]
        ),
    )
    def count_kernel(edge_hbm, zeros_hbm, ones_hbm, out_hbm, *scratch):
        didx = scratch[:DEPTH]
        onesv, stagev, accsh = scratch[DEPTH:DEPTH + 3]
        sems = scratch[DEPTH + 3:]
        cid = lax.axis_index("c")
        sid = lax.axis_index("s")
        wid = sid * NC + cid
        pltpu.sync_copy(zeros_hbm, stagev)
        pltpu.sync_copy(stagev, accsh.at[pl.ds(sid * slice_len, slice_len)])
        pltpu.sync_copy(ones_hbm, onesv)
        plsc.subcore_barrier()
        base = E + wid * epw
        pending = [None] * DEPTH
        for i in range(nchunks):
            r = i % DEPTH
            if pending[r] is not None:
                pending[r].wait()
            pltpu.sync_copy(edge_hbm.at[pl.ds(base + i * chunk, chunk)], didx[r])
            pending[r] = pltpu.async_copy(onesv, accsh.at[didx[r]], sems[r], add=True)
        for p in pending:
            if p is not None:
                p.wait()
        plsc.subcore_barrier()
        pltpu.sync_copy(accsh.at[pl.ds(sid * slice_len, slice_len)], stagev)
        pltpu.sync_copy(stagev, out_hbm.at[pl.ds(cid * n_pad + sid * slice_len, slice_len)])

    return count_kernel


def _make_sc_gather_scatter(E, n_pad, slice_len, chunk):
    epw = E // NW
    nchunks = epw // chunk

    @functools.partial(
        pl.kernel,
        out_type=jax.ShapeDtypeStruct((NC * n_pad,), jnp.float32),
        mesh=_mesh(),
        scratch_types=(
            [pltpu.VMEM((chunk,), jnp.int32) for _ in range(2 * DEPTH)]
            + [pltpu.VMEM((chunk,), jnp.float32) for _ in range(DEPTH)]
            + [
                pltpu.VMEM((slice_len,), jnp.float32),
                pltpu.VMEM_SHARED((n_pad,), jnp.float32),
                pltpu.VMEM_SHARED((n_pad,), jnp.float32),
            ]
            + [pltpu.SemaphoreType.DMA for _ in range(2 * DEPTH)]
        ),
    )
    def gs_kernel(edge_hbm, table_hbm, zeros_hbm, out_hbm, *scratch):
        sidx = scratch[:DEPTH]
        didx = scratch[DEPTH:2 * DEPTH]
        vals = scratch[2 * DEPTH:3 * DEPTH]
        stagev, tabsh, accsh = scratch[3 * DEPTH:3 * DEPTH + 3]
        semg = scratch[3 * DEPTH + 3:3 * DEPTH + 3 + DEPTH]
        sems = scratch[3 * DEPTH + 3 + DEPTH:]
        cid = lax.axis_index("c")
        sid = lax.axis_index("s")
        wid = sid * NC + cid
        sl = pl.ds(sid * slice_len, slice_len)
        pltpu.sync_copy(zeros_hbm, stagev)
        pltpu.sync_copy(stagev, accsh.at[sl])
        pltpu.sync_copy(table_hbm.at[sl], stagev)
        pltpu.sync_copy(stagev, tabsh.at[sl])
        plsc.subcore_barrier()
        base = wid * epw
        gat = [None] * DEPTH
        pending = [None] * DEPTH
        for i in range(nchunks):
            r = i % DEPTH
            if pending[r] is not None:
                pending[r].wait()
                pending[r] = None
            off = base + i * chunk
            pltpu.sync_copy(edge_hbm.at[pl.ds(off, chunk)], sidx[r])
            pltpu.sync_copy(edge_hbm.at[pl.ds(E + off, chunk)], didx[r])
            gat[r] = pltpu.async_copy(tabsh.at[sidx[r]], vals[r], semg[r])
            if i >= 1:
                rp = (i - 1) % DEPTH
                gat[rp].wait()
                gat[rp] = None
                pending[rp] = pltpu.async_copy(
                    vals[rp], accsh.at[didx[rp]], sems[rp], add=True)
        rl = (nchunks - 1) % DEPTH
        gat[rl].wait()
        pending[rl] = pltpu.async_copy(vals[rl], accsh.at[didx[rl]], sems[rl], add=True)
        for p in pending:
            if p is not None:
                p.wait()
        plsc.subcore_barrier()
        pltpu.sync_copy(accsh.at[sl], stagev)
        pltpu.sync_copy(stagev, out_hbm.at[pl.ds(cid * n_pad + sid * slice_len, slice_len)])

    return gs_kernel


def _tc1_body(cnt_ref, xp_ref, dinv_ref, u1_ref):
    deg = cnt_ref[0] + cnt_ref[1] + 1.0
    dinv = lax.rsqrt(deg)
    dinv_ref[...] = dinv
    u1_ref[...] = dinv * xp_ref[...]


def _tc2_body(t1_ref, dinv_ref, xp_ref, w1_ref, b1_ref, w2_ref, y_ref, u2_ref):
    dinv = dinv_ref[...]
    s = dinv * (t1_ref[0] + t1_ref[1]) + dinv * dinv * xp_ref[...]
    y = jnp.zeros_like(s)
    for j in range(16):
        y = y + jnp.maximum(s * w1_ref[0, j] + b1_ref[0, j], 0.0) * w2_ref[0, j]
    y_ref[...] = y
    u2_ref[...] = dinv * y


def _tc3_body(t2_ref, dinv_ref, y_ref, b2_ref, out_ref):
    dinv = dinv_ref[...]
    out_ref[...] = dinv * (t2_ref[0] + t2_ref[1]) + dinv * dinv * y_ref[...] + b2_ref[0, 0]


def kernel(x, edge_index, W1, b1, W2, b2):
    N = x.shape[0]
    E = edge_index.shape[1]
    assert E % NW == 0
    epw = E // NW
    chunk_cnt = 10000
    chunk_gs = 5000
    assert epw % chunk_cnt == 0 and epw % chunk_gs == 0
    slice_len = (-(-N // NS) + 7) // 8 * 8
    n_pad = slice_len * NS
    rows = n_pad // 128

    sc_count = _make_sc_count(E, n_pad, slice_len, chunk_cnt)
    sc_gs = _make_sc_gather_scatter(E, n_pad, slice_len, chunk_gs)

    zeros_s = jnp.zeros((slice_len,), jnp.float32)
    ones_c = jnp.ones((chunk_cnt,), jnp.float32)
    xp = jnp.pad(x[:, 0], (0, n_pad - N)).reshape(rows, 128)

    ei_flat = edge_index.reshape(2 * E)
    cnt = sc_count(ei_flat, zeros_s, ones_c)

    dinv, u1 = pl.pallas_call(
        _tc1_body,
        out_shape=[
            jax.ShapeDtypeStruct((rows, 128), jnp.float32),
            jax.ShapeDtypeStruct((rows, 128), jnp.float32),
        ],
    )(cnt.reshape(2, rows, 128), xp)

    t1 = sc_gs(ei_flat, u1.reshape(n_pad), zeros_s)

    y, u2 = pl.pallas_call(
        _tc2_body,
        out_shape=[
            jax.ShapeDtypeStruct((rows, 128), jnp.float32),
            jax.ShapeDtypeStruct((rows, 128), jnp.float32),
        ],
    )(t1.reshape(2, rows, 128), dinv, xp, W1, b1.reshape(1, 16), W2.reshape(1, 16))

    t2 = sc_gs(ei_flat, u2.reshape(n_pad), zeros_s)

    out = pl.pallas_call(
        _tc3_body,
        out_shape=jax.ShapeDtypeStruct((rows, 128), jnp.float32),
    )(t2.reshape(2, rows, 128), dinv, y, b2.reshape(1, 1))

    return out.reshape(n_pad)[:N].reshape(N, 1)
